# baseline (speedup 1.0000x reference)
import functools
import math

import jax
import jax.numpy as jnp
from jax import lax
from jax.experimental import pallas as pl
from jax.experimental.pallas import tpu as pltpu
from jax.experimental.pallas import tpu_sc as plsc

D_MODEL = 64
SCALE = math.sqrt(D_MODEL)
LANES = 16

NUM_CORES = 2
NUM_SUBCORES = 16
NUM_WORKERS = NUM_CORES * NUM_SUBCORES

CHUNK = 128


@functools.partial(jax.jit, static_argnames=("n_per_w", "n_chunks"))
def _embed_sc(x3d, weight, *, n_per_w, n_chunks):
    n_total = NUM_WORKERS * n_per_w
    mesh = plsc.VectorSubcoreMesh(core_axis_name="c", subcore_axis_name="s")

    @functools.partial(
        pl.kernel,
        out_type=jax.ShapeDtypeStruct((n_total, D_MODEL), jnp.float32),
        mesh=mesh,
        scratch_types=[
            pltpu.VMEM((n_chunks, CHUNK), jnp.int32),
            pltpu.VMEM((CHUNK, D_MODEL), jnp.float32),
            pltpu.SemaphoreType.DMA,
        ],
        compiler_params=pltpu.CompilerParams(use_tc_tiling_on_sc=False),
    )
    def body(w_hbm, idx_hbm, out_hbm, idx_v, rows_v, gsem):
        wid = lax.axis_index("s") * NUM_CORES + lax.axis_index("c")
        base = wid * n_per_w
        pltpu.sync_copy(idx_hbm.at[wid], idx_v)

        @pl.loop(0, n_chunks)
        def chunk_loop(c):
            pltpu.async_copy(w_hbm.at[idx_v.at[c]], rows_v, gsem).wait()

            @pl.loop(0, CHUNK)
            def row_loop(i):
                for j in range(D_MODEL // LANES):
                    sl = pl.ds(j * LANES, LANES)
                    rows_v[i, sl] = rows_v[i, sl] * SCALE

            pltpu.sync_copy(rows_v, out_hbm.at[pl.ds(base + c * CHUNK, CHUNK)])

    return body(weight, x3d)


def kernel(x, weight):
    b, t = x.shape
    n = b * t
    assert n % (NUM_WORKERS * CHUNK) == 0
    n_per_w = n // NUM_WORKERS
    n_chunks = n_per_w // CHUNK
    x3d = x.reshape(NUM_WORKERS, n_chunks, CHUNK).astype(jnp.int32)
    out = _embed_sc(x3d, weight, n_per_w=n_per_w, n_chunks=n_chunks)
    return out.reshape(b, t, D_MODEL)
